# baseline (speedup 1.0000x reference)
import jax
import jax.numpy as jnp
from jax import lax
from jax.experimental import pallas as pl
from jax.experimental.pallas import tpu as pltpu
from jax.experimental.pallas import tpu_sc as plsc

N_NODES = 10000
N_EDGES = 320000
D = 128
DG = D // 16
NC = 2
NS = 16
NW = NC * NS
EDGES_PER_W = N_EDGES // NW
CHUNK = 80
NCHUNK = EDGES_PER_W // CHUNK
NBUF = 4
NSLOT = 8
ILEAD = 6
GLEAD = NBUF - 1
N_PAD = 10240
ROWS_PER_SUB = N_PAD // NS
OUT_PER_W = N_PAD // NW
HALF = N_PAD // NC
MAGIC = 0x5CA1AB1E


def _sc_body(x_hbm, meta_hbm, trend_hbm,
             out_hbm, exch_hbm, agg1_hbm, flags_hbm,
             acc_sh, meta_r, trend_r, bufs, fbuf, idb,
             gsems, ssems, isems):
    cid = lax.axis_index("c")
    sid = lax.axis_index("s")
    wid = cid * NS + sid
    oth = 1 - cid

    def fill_zero_buf():
        def zf(i, _):
            r = i // DG
            c = (i % DG) * 16
            bufs[0, r, pl.ds(c, 16)] = jnp.zeros((16,), jnp.float32)
            return 0
        lax.fori_loop(0, CHUNK * DG, zf, 0)

    def zero_acc_slab():
        for t in range(ROWS_PER_SUB // CHUNK):
            pltpu.sync_copy(
                bufs.at[0],
                acc_sh.at[pl.ds(sid * ROWS_PER_SUB + t * CHUNK, CHUNK), :])

    def i_start(k):
        s = lax.rem(k, NSLOT)
        pltpu.async_copy(meta_hbm.at[wid, k], meta_r.at[s], isems.at[s])
        pltpu.async_copy(trend_hbm.at[wid, k], trend_r.at[s], isems.at[s])

    def i_wait(k):
        s = lax.rem(k, NSLOT)
        pltpu.make_async_copy(meta_hbm.at[wid, 0], meta_r.at[s],
                              isems.at[s]).wait()
        pltpu.make_async_copy(trend_hbm.at[wid, 0], trend_r.at[s],
                              isems.at[s]).wait()

    def pipeline(tab):
        def g_start(k):
            b = lax.rem(k, NBUF)
            s = lax.rem(k, NSLOT)
            pltpu.async_copy(tab.at[meta_r.at[s, 0]], bufs.at[b], gsems.at[b])

        def g_wait(k):
            b = lax.rem(k, NBUF)
            pltpu.make_async_copy(tab.at[meta_r.at[0, 0]], bufs.at[b],
                                  gsems.at[b]).wait()

        def s_start(k):
            b = lax.rem(k, NBUF)
            s = lax.rem(k, NSLOT)
            pltpu.async_copy(bufs.at[b], acc_sh.at[meta_r.at[s, 1]],
                             ssems.at[b], add=True)

        def s_wait(k):
            b = lax.rem(k, NBUF)
            pltpu.make_async_copy(bufs.at[b], acc_sh.at[meta_r.at[0, 1]],
                                  ssems.at[b]).wait()

        def compute(k, b):
            s = lax.rem(k, NSLOT)

            def group(g, _):
                t16 = trend_r[s, pl.ds(g * 16, 16)]
                for i in range(16):
                    tv = jnp.broadcast_to(t16[i], (16,))
                    e = g * 16 + i
                    for j in range(DG):
                        bufs[b, e, pl.ds(j * 16, 16)] = (
                            bufs[b, e, pl.ds(j * 16, 16)] * tv)
                return 0
            lax.fori_loop(0, CHUNK // 16, group, 0)

        for kk in range(ILEAD):
            i_start(kk)
        for kk in range(GLEAD):
            i_wait(kk)
            g_start(kk)

        def pipe(i, _):
            for b in range(NBUF):
                k = i * NBUF + b
                g_wait(k)
                compute(k, b)

                @pl.when(k > 0)
                def _():
                    s_wait(k - 1)

                @pl.when(k < NCHUNK - ILEAD)
                def _():
                    i_start(k + ILEAD)

                @pl.when(k < NCHUNK - GLEAD)
                def _():
                    i_wait(k + GLEAD)
                    g_start(k + GLEAD)

                s_start(k)
            return 0
        lax.fori_loop(0, NCHUNK // NBUF, pipe, 0)
        for k in range(NCHUNK - (NCHUNK % NBUF), NCHUNK):
            g_wait(k)
            compute(k, k % NBUF)
            s_wait(k - 1)
            s_start(k)
        s_wait(NCHUNK - 1)

    def publish_and_sync(phase, lo_row, n_rows):
        for t in range(n_rows // CHUNK):
            r0 = lo_row + t * CHUNK
            pltpu.sync_copy(acc_sh.at[pl.ds(r0, CHUNK), :],
                            exch_hbm.at[cid, pl.ds(r0, CHUNK), :])
        plsc.subcore_barrier()

        @pl.when(sid == 0)
        def _():
            fbuf[...] = jnp.full((16,), MAGIC, jnp.int32)
            pltpu.sync_copy(fbuf, flags_hbm.at[cid, phase])

        fbuf[...] = jnp.zeros((16,), jnp.int32)

        def poll(i, _):
            @pl.when(fbuf[...][0] != MAGIC)
            def _():
                pltpu.sync_copy(flags_hbm.at[oth, phase], fbuf)
            return 0
        lax.fori_loop(0, 256, poll, 0)

    def add_peer(lo_row, n_rows):
        for t in range(n_rows // CHUNK):
            r0 = lo_row + t * CHUNK
            pltpu.sync_copy(exch_hbm.at[oth, pl.ds(r0, CHUNK), :], bufs.at[0])
            for j in range(CHUNK // 16):
                idb[pl.ds(j * 16, 16)] = (
                    jax.lax.broadcasted_iota(jnp.int32, (16,), 0)
                    + (r0 + j * 16))
            pltpu.sync_copy(bufs.at[0], acc_sh.at[idb], add=True)

    @pl.when(sid == 0)
    def _():
        fbuf[...] = jnp.zeros((16,), jnp.int32)
        pltpu.sync_copy(fbuf, flags_hbm.at[cid, 0])
        pltpu.sync_copy(fbuf, flags_hbm.at[cid, 1])

    fill_zero_buf()
    zero_acc_slab()
    plsc.subcore_barrier()
    pipeline(x_hbm)
    plsc.subcore_barrier()

    publish_and_sync(0, sid * ROWS_PER_SUB, ROWS_PER_SUB)
    add_peer(sid * ROWS_PER_SUB, ROWS_PER_SUB)

    for t in range(ROWS_PER_SUB // CHUNK):
        r0 = sid * ROWS_PER_SUB + t * CHUNK
        pltpu.sync_copy(acc_sh.at[pl.ds(r0, CHUNK), :],
                        agg1_hbm.at[cid, pl.ds(r0, CHUNK), :])
    fill_zero_buf()
    zero_acc_slab()
    plsc.subcore_barrier()
    pipeline(agg1_hbm.at[cid])
    plsc.subcore_barrier()

    publish_and_sync(1, oth * HALF + sid * (HALF // NS), HALF // NS)
    out_lo = wid * OUT_PER_W
    add_peer(out_lo, OUT_PER_W)

    def final_chunk(r0):
        pltpu.sync_copy(x_hbm.at[pl.ds(r0, CHUNK), :], bufs.at[1])
        pltpu.sync_copy(agg1_hbm.at[cid, pl.ds(r0, CHUNK), :], bufs.at[2])
        pltpu.sync_copy(acc_sh.at[pl.ds(r0, CHUNK), :], bufs.at[3])

        def frow(e, _):
            for j in range(DG):
                sl = pl.ds(j * 16, 16)
                bufs[1, e, sl] = (
                    (bufs[1, e, sl] + bufs[2, e, sl] + bufs[3, e, sl])
                    * jnp.float32(1.0 / 3.0))
            return 0
        lax.fori_loop(0, CHUNK, frow, 0)
        pltpu.sync_copy(bufs.at[1], out_hbm.at[pl.ds(r0, CHUNK), :])

    @pl.when(wid < NW - 1)
    def _():
        for t in range(OUT_PER_W // CHUNK):
            final_chunk(out_lo + t * CHUNK)

    @pl.when(wid == NW - 1)
    def _():
        final_chunk(out_lo)


_sc_all = pl.kernel(
    _sc_body,
    out_type=(
        jax.ShapeDtypeStruct((N_PAD, D), jnp.float32),
        jax.ShapeDtypeStruct((NC, N_PAD, D), jnp.float32),
        jax.ShapeDtypeStruct((NC, N_PAD, D), jnp.float32),
        jax.ShapeDtypeStruct((NC, 2, 16), jnp.int32),
    ),
    mesh=plsc.VectorSubcoreMesh(core_axis_name="c", subcore_axis_name="s"),
    scratch_types=[
        pltpu.VMEM_SHARED((N_PAD, D), jnp.float32),
        pltpu.VMEM((NSLOT, 2, CHUNK), jnp.int32),
        pltpu.VMEM((NSLOT, CHUNK), jnp.float32),
        pltpu.VMEM((NBUF, CHUNK, D), jnp.float32),
        pltpu.VMEM((16,), jnp.int32),
        pltpu.VMEM((CHUNK,), jnp.int32),
        pltpu.SemaphoreType.DMA((NBUF,)),
        pltpu.SemaphoreType.DMA((NBUF,)),
        pltpu.SemaphoreType.DMA((NSLOT,)),
    ],
)


def kernel(embed, edge_index, trend):
    row = edge_index[0].astype(jnp.int32).reshape(NW, NCHUNK, 1, CHUNK)
    col = edge_index[1].astype(jnp.int32).reshape(NW, NCHUNK, 1, CHUNK)
    meta = jnp.concatenate([row, col], axis=2)
    trendr = trend.astype(jnp.float32).reshape(NW, NCHUNK, CHUNK)

    out_pad, _, _, _ = _sc_all(embed, meta, trendr)
    return out_pad[:N_NODES]


# speedup vs baseline: 1.0650x; 1.0650x over previous
import jax
import jax.numpy as jnp
from jax import lax
from jax.experimental import pallas as pl
from jax.experimental.pallas import tpu as pltpu
from jax.experimental.pallas import tpu_sc as plsc

N_NODES = 10000
N_EDGES = 320000
D = 128
DG = D // 16
NC = 2
NS = 16
NW = NC * NS
EDGES_PER_W = N_EDGES // NW
CHUNK = 80
NCHUNK = EDGES_PER_W // CHUNK
NBUF = 4
NSLOT = 8
ILEAD = 6
GLEAD = NBUF - 1
N_PAD = 10240
ROWS_PER_SUB = N_PAD // NS
OUT_PER_W = N_PAD // NW
HALF = N_PAD // NC
MAGIC = 0x5CA1AB1E


def _sc_body(x_hbm, meta_hbm, trend_hbm,
             out_hbm, exch_hbm, agg1_hbm, flags_hbm,
             acc_sh, meta_r, trend_r, bufs, fbuf, idb,
             gsems, ssems, isems):
    cid = lax.axis_index("c")
    sid = lax.axis_index("s")
    wid = cid * NS + sid
    oth = 1 - cid

    def fill_zero_buf():
        def zf(i, _):
            r = i // DG
            c = (i % DG) * 16
            bufs[0, r, pl.ds(c, 16)] = jnp.zeros((16,), jnp.float32)
            return 0
        lax.fori_loop(0, CHUNK * DG, zf, 0)

    def zero_acc_slab():
        for t in range(ROWS_PER_SUB // CHUNK):
            pltpu.sync_copy(
                bufs.at[0],
                acc_sh.at[pl.ds(sid * ROWS_PER_SUB + t * CHUNK, CHUNK), :])

    def i_start(k):
        s = lax.rem(k, NSLOT)
        pltpu.async_copy(meta_hbm.at[wid, k], meta_r.at[s], isems.at[s])
        pltpu.async_copy(trend_hbm.at[wid, k], trend_r.at[s], isems.at[s])

    def i_wait(k):
        s = lax.rem(k, NSLOT)
        pltpu.make_async_copy(meta_hbm.at[wid, 0], meta_r.at[s],
                              isems.at[s]).wait()
        pltpu.make_async_copy(trend_hbm.at[wid, 0], trend_r.at[s],
                              isems.at[s]).wait()

    def pipeline(tab):
        def g_start(k):
            b = lax.rem(k, NBUF)
            s = lax.rem(k, NSLOT)
            pltpu.async_copy(tab.at[meta_r.at[s, 0]], bufs.at[b], gsems.at[b])

        def g_wait(k):
            b = lax.rem(k, NBUF)
            pltpu.make_async_copy(tab.at[meta_r.at[0, 0]], bufs.at[b],
                                  gsems.at[b]).wait()

        def s_start(k):
            b = lax.rem(k, NBUF)
            s = lax.rem(k, NSLOT)
            pltpu.async_copy(bufs.at[b], acc_sh.at[meta_r.at[s, 1]],
                             ssems.at[b], add=True)

        def s_wait(k):
            b = lax.rem(k, NBUF)
            pltpu.make_async_copy(bufs.at[b], acc_sh.at[meta_r.at[0, 1]],
                                  ssems.at[b]).wait()

        def compute(k, b):
            s = lax.rem(k, NSLOT)

            def group(g, _):
                t16 = trend_r[s, pl.ds(g * 16, 16)]
                for i in range(16):
                    tv = jnp.broadcast_to(t16[i], (16,))
                    e = g * 16 + i
                    for j in range(DG):
                        bufs[b, e, pl.ds(j * 16, 16)] = (
                            bufs[b, e, pl.ds(j * 16, 16)] * tv)
                return 0
            lax.fori_loop(0, CHUNK // 16, group, 0)

        for kk in range(ILEAD):
            i_start(kk)
        for kk in range(GLEAD):
            i_wait(kk)
            g_start(kk)

        def pipe(i, _):
            for b in range(NBUF):
                k = i * NBUF + b
                g_wait(k)
                compute(k, b)

                @pl.when(k > 0)
                def _():
                    s_wait(k - 1)

                @pl.when(k < NCHUNK - ILEAD)
                def _():
                    i_start(k + ILEAD)

                @pl.when(k < NCHUNK - GLEAD)
                def _():
                    i_wait(k + GLEAD)
                    g_start(k + GLEAD)

                s_start(k)
            return 0
        lax.fori_loop(0, NCHUNK // NBUF, pipe, 0)
        for k in range(NCHUNK - (NCHUNK % NBUF), NCHUNK):
            g_wait(k)
            compute(k, k % NBUF)
            s_wait(k - 1)
            s_start(k)
        s_wait(NCHUNK - 1)

    def publish_and_sync(phase, lo_row, n_rows):
        nt = n_rows // CHUNK
        for t in range(nt):
            r0 = lo_row + t * CHUNK
            pltpu.async_copy(acc_sh.at[pl.ds(r0, CHUNK), :],
                             exch_hbm.at[cid, pl.ds(r0, CHUNK), :],
                             isems.at[t])
        for t in range(nt):
            r0 = lo_row + t * CHUNK
            pltpu.make_async_copy(acc_sh.at[pl.ds(r0, CHUNK), :],
                                  exch_hbm.at[cid, pl.ds(r0, CHUNK), :],
                                  isems.at[t]).wait()
        plsc.subcore_barrier()

        @pl.when(sid == 0)
        def _():
            fbuf[...] = jnp.full((16,), MAGIC, jnp.int32)
            pltpu.sync_copy(fbuf, flags_hbm.at[cid, phase])

        fbuf[...] = jnp.zeros((16,), jnp.int32)

        def poll(i, _):
            @pl.when(fbuf[...][0] != MAGIC)
            def _():
                pltpu.sync_copy(flags_hbm.at[oth, phase], fbuf)
            return 0
        lax.fori_loop(0, 256, poll, 0)

    def build_agg1(lo_row, n_rows):
        nt = n_rows // CHUNK

        def ra(k, b):
            return (exch_hbm.at[oth, pl.ds(lo_row + k * CHUNK, CHUNK), :],
                    bufs.at[b], gsems.at[b])

        def rb(k, b):
            return (acc_sh.at[pl.ds(lo_row + k * CHUNK, CHUNK), :],
                    bufs.at[2 + b], ssems.at[b])

        def wr(k, b):
            return (bufs.at[b],
                    agg1_hbm.at[cid, pl.ds(lo_row + k * CHUNK, CHUNK), :],
                    isems.at[k])

        for k in range(2):
            pltpu.async_copy(*ra(k, k))
            pltpu.async_copy(*rb(k, k))
        for k in range(nt):
            b = k % 2
            pltpu.make_async_copy(*ra(k, b)).wait()
            pltpu.make_async_copy(*rb(k, b)).wait()

            def vadd(e, _):
                for j in range(DG):
                    sl = pl.ds(j * 16, 16)
                    bufs[b, e, sl] = bufs[b, e, sl] + bufs[2 + b, e, sl]
                return 0
            lax.fori_loop(0, CHUNK, vadd, 0)
            pltpu.async_copy(*wr(k, b))
            if k + 2 < nt:
                pltpu.make_async_copy(*wr(k, b)).wait()
                pltpu.async_copy(*ra(k + 2, b))
                pltpu.async_copy(*rb(k + 2, b))
        for k in range(max(nt - 2, 0), nt):
            pltpu.make_async_copy(*wr(k, k % 2)).wait()

    @pl.when(sid == 0)
    def _():
        fbuf[...] = jnp.zeros((16,), jnp.int32)
        pltpu.sync_copy(fbuf, flags_hbm.at[cid, 0])
        pltpu.sync_copy(fbuf, flags_hbm.at[cid, 1])

    fill_zero_buf()
    zero_acc_slab()
    plsc.subcore_barrier()
    pipeline(x_hbm)
    plsc.subcore_barrier()

    publish_and_sync(0, sid * ROWS_PER_SUB, ROWS_PER_SUB)
    build_agg1(sid * ROWS_PER_SUB, ROWS_PER_SUB)
    plsc.subcore_barrier()

    pipeline(agg1_hbm.at[cid])
    plsc.subcore_barrier()

    publish_and_sync(1, oth * HALF + sid * (HALF // NS), HALF // NS)
    out_lo = wid * OUT_PER_W

    def final_chunk(r0):
        pltpu.async_copy(x_hbm.at[pl.ds(r0, CHUNK), :], bufs.at[0],
                         gsems.at[0])
        pltpu.async_copy(exch_hbm.at[oth, pl.ds(r0, CHUNK), :], bufs.at[1],
                         gsems.at[1])
        pltpu.async_copy(acc_sh.at[pl.ds(r0, CHUNK), :], bufs.at[2],
                         gsems.at[2])
        pltpu.make_async_copy(x_hbm.at[pl.ds(r0, CHUNK), :], bufs.at[0],
                              gsems.at[0]).wait()
        pltpu.make_async_copy(exch_hbm.at[oth, pl.ds(r0, CHUNK), :],
                              bufs.at[1], gsems.at[1]).wait()
        pltpu.make_async_copy(acc_sh.at[pl.ds(r0, CHUNK), :], bufs.at[2],
                              gsems.at[2]).wait()

        def frow(e, _):
            for j in range(DG):
                sl = pl.ds(j * 16, 16)
                bufs[0, e, sl] = (
                    (bufs[0, e, sl] + bufs[1, e, sl] + bufs[2, e, sl])
                    * jnp.float32(1.0 / 3.0))
            return 0
        lax.fori_loop(0, CHUNK, frow, 0)
        pltpu.sync_copy(bufs.at[0], out_hbm.at[pl.ds(r0, CHUNK), :])

    @pl.when(wid < NW - 1)
    def _():
        for t in range(OUT_PER_W // CHUNK):
            final_chunk(out_lo + t * CHUNK)

    @pl.when(wid == NW - 1)
    def _():
        final_chunk(out_lo)


_sc_all = pl.kernel(
    _sc_body,
    out_type=(
        jax.ShapeDtypeStruct((N_PAD, D), jnp.float32),
        jax.ShapeDtypeStruct((NC, N_PAD, D), jnp.float32),
        jax.ShapeDtypeStruct((NC, N_PAD, D), jnp.float32),
        jax.ShapeDtypeStruct((NC, 2, 16), jnp.int32),
    ),
    mesh=plsc.VectorSubcoreMesh(core_axis_name="c", subcore_axis_name="s"),
    scratch_types=[
        pltpu.VMEM_SHARED((N_PAD, D), jnp.float32),
        pltpu.VMEM((NSLOT, 2, CHUNK), jnp.int32),
        pltpu.VMEM((NSLOT, CHUNK), jnp.float32),
        pltpu.VMEM((NBUF, CHUNK, D), jnp.float32),
        pltpu.VMEM((16,), jnp.int32),
        pltpu.VMEM((CHUNK,), jnp.int32),
        pltpu.SemaphoreType.DMA((NBUF,)),
        pltpu.SemaphoreType.DMA((NBUF,)),
        pltpu.SemaphoreType.DMA((NSLOT,)),
    ],
)


def kernel(embed, edge_index, trend):
    row = edge_index[0].astype(jnp.int32).reshape(NW, NCHUNK, 1, CHUNK)
    col = edge_index[1].astype(jnp.int32).reshape(NW, NCHUNK, 1, CHUNK)
    meta = jnp.concatenate([row, col], axis=2)
    trendr = trend.astype(jnp.float32).reshape(NW, NCHUNK, CHUNK)

    out_pad, _, _, _ = _sc_all(embed, meta, trendr)
    return out_pad[:N_NODES]


# speedup vs baseline: 1.0881x; 1.0217x over previous
import jax
import jax.numpy as jnp
from jax import lax
from jax.experimental import pallas as pl
from jax.experimental.pallas import tpu as pltpu
from jax.experimental.pallas import tpu_sc as plsc

N_NODES = 10000
N_EDGES = 320000
D = 128
NC = 2
NS = 16
NW = NC * NS
EDGES_PER_W = N_EDGES // NW
CHUNK = 80
NCHUNK = EDGES_PER_W // CHUNK
NBUF = 4
NSLOT = 8
ILEAD = 6
GLEAD = NBUF - 1
N_PAD = 10240
ROWS_PER_SUB = N_PAD // NS


def _sc_layer_body(x_hbm, meta_hbm, trend_hbm, out_hbm, acc_sh, meta_r,
                   trend_r, bufs, gsems, ssems, isems):
    cid = lax.axis_index("c")
    sid = lax.axis_index("s")
    wid = cid * NS + sid

    def zfill(i, _):
        r = i // (D // 16)
        c = (i % (D // 16)) * 16
        bufs[0, r, pl.ds(c, 16)] = jnp.zeros((16,), jnp.float32)
        return 0
    lax.fori_loop(0, CHUNK * (D // 16), zfill, 0)
    for t in range(ROWS_PER_SUB // CHUNK):
        pltpu.sync_copy(bufs.at[0],
                        acc_sh.at[pl.ds(sid * ROWS_PER_SUB + t * CHUNK, CHUNK), :])
    plsc.subcore_barrier()

    def i_start(k):
        s = lax.rem(k, NSLOT)
        pltpu.async_copy(meta_hbm.at[wid, k], meta_r.at[s], isems.at[s])
        pltpu.async_copy(trend_hbm.at[wid, k], trend_r.at[s], isems.at[s])

    def i_wait(k):
        s = lax.rem(k, NSLOT)
        pltpu.make_async_copy(meta_hbm.at[wid, 0], meta_r.at[s],
                              isems.at[s]).wait()
        pltpu.make_async_copy(trend_hbm.at[wid, 0], trend_r.at[s],
                              isems.at[s]).wait()

    def g_start(k):
        b = lax.rem(k, NBUF)
        s = lax.rem(k, NSLOT)
        pltpu.async_copy(x_hbm.at[meta_r.at[s, 0]], bufs.at[b], gsems.at[b])

    def g_wait(k):
        b = lax.rem(k, NBUF)
        pltpu.make_async_copy(x_hbm.at[meta_r.at[0, 0]], bufs.at[b],
                              gsems.at[b]).wait()

    def s_start(k):
        b = lax.rem(k, NBUF)
        s = lax.rem(k, NSLOT)
        pltpu.async_copy(bufs.at[b], acc_sh.at[meta_r.at[s, 1]],
                         ssems.at[b], add=True)

    def s_wait(k):
        b = lax.rem(k, NBUF)
        pltpu.make_async_copy(bufs.at[b], acc_sh.at[meta_r.at[0, 1]],
                              ssems.at[b]).wait()

    def compute(k, b):
        s = lax.rem(k, NSLOT)

        def group(g, _):
            t16 = trend_r[s, pl.ds(g * 16, 16)]
            for i in range(16):
                tv = jnp.broadcast_to(t16[i], (16,))
                e = g * 16 + i
                for j in range(D // 16):
                    bufs[b, e, pl.ds(j * 16, 16)] = (
                        bufs[b, e, pl.ds(j * 16, 16)] * tv)
            return 0
        lax.fori_loop(0, CHUNK // 16, group, 0)

    for kk in range(ILEAD):
        i_start(kk)
    for kk in range(GLEAD):
        i_wait(kk)
        g_start(kk)

    def pipe(i, _):
        for b in range(NBUF):
            k = i * NBUF + b
            g_wait(k)
            compute(k, b)

            @pl.when(k > 0)
            def _():
                s_wait(k - 1)

            @pl.when(k < NCHUNK - ILEAD)
            def _():
                i_start(k + ILEAD)

            @pl.when(k < NCHUNK - GLEAD)
            def _():
                i_wait(k + GLEAD)
                g_start(k + GLEAD)

            s_start(k)
        return 0
    lax.fori_loop(0, NCHUNK // NBUF, pipe, 0)
    kl = NCHUNK - (NCHUNK % NBUF)
    for k in range(kl, NCHUNK):
        g_wait(k)
        compute(k, k % NBUF)
        s_wait(k - 1)
        s_start(k)
    s_wait(NCHUNK - 1)

    plsc.subcore_barrier()
    pltpu.sync_copy(acc_sh.at[pl.ds(sid * ROWS_PER_SUB, ROWS_PER_SUB), :],
                    out_hbm.at[cid, pl.ds(sid * ROWS_PER_SUB, ROWS_PER_SUB), :])


_sc_layer = pl.kernel(
    _sc_layer_body,
    out_type=jax.ShapeDtypeStruct((NC, N_PAD, D), jnp.float32),
    mesh=plsc.VectorSubcoreMesh(core_axis_name="c", subcore_axis_name="s"),
    scratch_types=[
        pltpu.VMEM_SHARED((N_PAD, D), jnp.float32),
        pltpu.VMEM((NSLOT, 2, CHUNK), jnp.int32),
        pltpu.VMEM((NSLOT, CHUNK), jnp.float32),
        pltpu.VMEM((NBUF, CHUNK, D), jnp.float32),
        pltpu.SemaphoreType.DMA((NBUF,)),
        pltpu.SemaphoreType.DMA((NBUF,)),
        pltpu.SemaphoreType.DMA((NSLOT,)),
    ],
)


def _add2_body(a_ref, b_ref, o_ref):
    o_ref[...] = a_ref[0] + b_ref[0]


def _add2(p):
    blk = 1000
    return pl.pallas_call(
        _add2_body,
        grid=(N_NODES // blk,),
        in_specs=[
            pl.BlockSpec((1, blk, D), lambda i: (0, i, 0)),
            pl.BlockSpec((1, blk, D), lambda i: (1, i, 0)),
        ],
        out_specs=pl.BlockSpec((blk, D), lambda i: (i, 0)),
        out_shape=jax.ShapeDtypeStruct((N_NODES, D), jnp.float32),
    )(p, p)


def _final_body(e_ref, a1_ref, p0_ref, p1_ref, o_ref):
    o_ref[...] = (e_ref[...] + a1_ref[...]
                  + p0_ref[0] + p1_ref[0]) * jnp.float32(1.0 / 3.0)


def _final(embed, agg1, p2):
    blk = 1000
    return pl.pallas_call(
        _final_body,
        grid=(N_NODES // blk,),
        in_specs=[
            pl.BlockSpec((blk, D), lambda i: (i, 0)),
            pl.BlockSpec((blk, D), lambda i: (i, 0)),
            pl.BlockSpec((1, blk, D), lambda i: (0, i, 0)),
            pl.BlockSpec((1, blk, D), lambda i: (1, i, 0)),
        ],
        out_specs=pl.BlockSpec((blk, D), lambda i: (i, 0)),
        out_shape=jax.ShapeDtypeStruct((N_NODES, D), jnp.float32),
    )(embed, agg1, p2, p2)


def kernel(embed, edge_index, trend):
    row = edge_index[0].astype(jnp.int32).reshape(NW, NCHUNK, 1, CHUNK)
    col = edge_index[1].astype(jnp.int32).reshape(NW, NCHUNK, 1, CHUNK)
    meta = jnp.concatenate([row, col], axis=2)
    trend = trend.astype(jnp.float32).reshape(NW, NCHUNK, CHUNK)

    p1 = _sc_layer(embed, meta, trend)
    agg1 = _add2(p1)
    p2 = _sc_layer(agg1, meta, trend)
    return _final(embed, agg1, p2)
